# Initial kernel scaffold; baseline (speedup 1.0000x reference)
import jax
import jax.numpy as jnp
from jax.experimental import pallas as pl


def kernel(edge_index, x_l, h0, c0, W_ih, W_hh, b_ih, b_hh):
    raise NotImplementedError("write your pallas kernel here")


# speedup vs baseline: 3.8521x; 3.8521x over previous
import functools

import jax
import jax.numpy as jnp
from jax import lax
from jax.experimental import pallas as pl
from jax.experimental.pallas import tpu as pltpu
from jax.experimental.pallas import tpu_sc as plsc

D = 128
N = 10000
E = 320000
NC, NS = 2, 16
NW = NC * NS
K = 128
CPW = -(-E // (K * NW))
EPAD = CPW * K * NW
ROWS_PER_TILE = 640
NPAD = NS * ROWS_PER_TILE


def _sc_aggregate_body(rowp, colp, xl, out, cidx_v, ridx_v, rows_v, z16_v,
                       msg_sh, sem):
    cid = lax.axis_index("c")
    sid = lax.axis_index("s")
    wid = cid * NS + sid

    zero = jnp.zeros((16,), jnp.float32)
    for i in range(16):
        for j in range(D // 16):
            z16_v[i, pl.ds(j * 16, 16)] = zero

    def zero_body(j, _):
        pltpu.sync_copy(z16_v, msg_sh.at[pl.ds(sid * ROWS_PER_TILE + j * 16, 16)])
        return _
    lax.fori_loop(0, ROWS_PER_TILE // 16, zero_body, None)
    plsc.subcore_barrier()

    def acc_body(j, _):
        base = (wid * CPW + j) * K
        pltpu.sync_copy(colp.at[pl.ds(base, K)], cidx_v)
        pltpu.sync_copy(rowp.at[pl.ds(base, K)], ridx_v)
        pltpu.async_copy(xl.at[cidx_v], rows_v, sem).wait()
        pltpu.sync_copy(rows_v, msg_sh.at[ridx_v], add=True)
        return _
    lax.fori_loop(0, CPW, acc_body, None)
    plsc.subcore_barrier()

    pltpu.sync_copy(msg_sh.at[pl.ds(sid * ROWS_PER_TILE, ROWS_PER_TILE)],
                    out.at[cid, pl.ds(sid * ROWS_PER_TILE, ROWS_PER_TILE)])


_sc_aggregate = functools.partial(
    pl.kernel,
    out_type=jax.ShapeDtypeStruct((NC, NPAD, D), jnp.float32),
    mesh=plsc.VectorSubcoreMesh(core_axis_name="c", subcore_axis_name="s",
                                num_cores=NC, num_subcores=NS),
    scratch_types=[
        pltpu.VMEM((K,), jnp.int32),
        pltpu.VMEM((K,), jnp.int32),
        pltpu.VMEM((K, D), jnp.float32),
        pltpu.VMEM((16, D), jnp.float32),
        pltpu.VMEM_SHARED((NPAD, D), jnp.float32),
        pltpu.SemaphoreType.DMA,
    ],
)(_sc_aggregate_body)


def _lstm_body(msgp_ref, h0_ref, c0_ref, wih_ref, whh_ref, bih_ref, bhh_ref,
               h_ref, c_ref):
    msg = msgp_ref[0] + msgp_ref[1]
    gates = (jnp.dot(msg, wih_ref[...], preferred_element_type=jnp.float32)
             + jnp.dot(h0_ref[...], whh_ref[...],
                       preferred_element_type=jnp.float32)
             + bih_ref[...] + bhh_ref[...])
    i = jax.nn.sigmoid(gates[:, 0 * D:1 * D])
    f = jax.nn.sigmoid(gates[:, 1 * D:2 * D])
    g = jnp.tanh(gates[:, 2 * D:3 * D])
    o = jax.nn.sigmoid(gates[:, 3 * D:4 * D])
    c_new = f * c0_ref[...] + i * g
    c_ref[...] = c_new
    h_ref[...] = o * jnp.tanh(c_new)


def _lstm_call(msgp, h0p, c0p, wihT, whhT, bih, bhh):
    R = 1024
    grid = NPAD // R
    return pl.pallas_call(
        _lstm_body,
        grid=(grid,),
        in_specs=[
            pl.BlockSpec((NC, R, D), lambda i: (0, i, 0)),
            pl.BlockSpec((R, D), lambda i: (i, 0)),
            pl.BlockSpec((R, D), lambda i: (i, 0)),
            pl.BlockSpec((D, 4 * D), lambda i: (0, 0)),
            pl.BlockSpec((D, 4 * D), lambda i: (0, 0)),
            pl.BlockSpec((1, 4 * D), lambda i: (0, 0)),
            pl.BlockSpec((1, 4 * D), lambda i: (0, 0)),
        ],
        out_specs=[
            pl.BlockSpec((R, D), lambda i: (i, 0)),
            pl.BlockSpec((R, D), lambda i: (i, 0)),
        ],
        out_shape=[
            jax.ShapeDtypeStruct((NPAD, D), jnp.float32),
            jax.ShapeDtypeStruct((NPAD, D), jnp.float32),
        ],
    )(msgp, h0p, c0p, wihT, whhT, bih, bhh)


def kernel(edge_index, x_l, h0, c0, W_ih, W_hh, b_ih, b_hh):
    ei = edge_index.astype(jnp.int32)
    pad = EPAD - E
    rowp = jnp.concatenate([ei[0], jnp.full((pad,), N, jnp.int32)])
    colp = jnp.concatenate([ei[1], jnp.zeros((pad,), jnp.int32)])

    msgp = _sc_aggregate(rowp, colp, x_l)

    h0p = jnp.pad(h0, ((0, NPAD - N), (0, 0)))
    c0p = jnp.pad(c0, ((0, NPAD - N), (0, 0)))
    h_new, c_new = _lstm_call(msgp, h0p, c0p, W_ih.T, W_hh.T,
                              b_ih.reshape(1, 4 * D), b_hh.reshape(1, 4 * D))
    return (h_new[:N], c_new[:N])


# speedup vs baseline: 4.9217x; 1.2777x over previous
import functools

import jax
import jax.numpy as jnp
from jax import lax
from jax.experimental import pallas as pl
from jax.experimental.pallas import tpu as pltpu
from jax.experimental.pallas import tpu_sc as plsc

D = 128
N = 10000
E = 320000
NC, NS = 2, 16
NW = NC * NS
K = 128
CPW = -(-E // (K * NW))
NCHUNK = CPW * NW
EPAD = NCHUNK * K
ROWS_PER_TILE = 640
NPAD = NS * ROWS_PER_TILE


def _sc_aggregate_body(rc, xl, out, idx0, idx1, rows0, rows1, z16_v,
                       msg_sh, isem0, isem1, gsem0, gsem1):
    cid = lax.axis_index("c")
    sid = lax.axis_index("s")
    wid = cid * NS + sid
    c0 = wid * CPW

    zero = jnp.zeros((16,), jnp.float32)
    for i in range(16):
        for j in range(D // 16):
            z16_v[i, pl.ds(j * 16, 16)] = zero

    def zero_body(j, carry):
        pltpu.sync_copy(z16_v, msg_sh.at[pl.ds(sid * ROWS_PER_TILE + j * 16, 16)])
        return carry
    lax.fori_loop(0, ROWS_PER_TILE // 16, zero_body, 0)
    plsc.subcore_barrier()

    pltpu.async_copy(rc.at[c0], idx0, isem0).wait()
    pltpu.async_copy(xl.at[idx0.at[0]], rows0, gsem0)
    pltpu.async_copy(rc.at[c0 + 1], idx1, isem1)

    def acc_body(t, carry):
        j = c0 + 2 * t
        pltpu.make_async_copy(xl.at[idx0.at[0]], rows0, gsem0).wait()
        pltpu.make_async_copy(rc.at[j + 1], idx1, isem1).wait()
        pltpu.async_copy(xl.at[idx1.at[0]], rows1, gsem1)
        pltpu.sync_copy(rows0, msg_sh.at[idx0.at[1]], add=True)
        pltpu.async_copy(rc.at[j + 2], idx0, isem0)
        pltpu.make_async_copy(xl.at[idx1.at[0]], rows1, gsem1).wait()
        pltpu.make_async_copy(rc.at[j + 2], idx0, isem0).wait()
        pltpu.async_copy(xl.at[idx0.at[0]], rows0, gsem0)
        pltpu.sync_copy(rows1, msg_sh.at[idx1.at[1]], add=True)
        pltpu.async_copy(rc.at[j + 3], idx1, isem1)
        return carry
    lax.fori_loop(0, (CPW - 3) // 2, acc_body, 0)

    jt = c0 + CPW - 3
    pltpu.make_async_copy(xl.at[idx0.at[0]], rows0, gsem0).wait()
    pltpu.make_async_copy(rc.at[jt + 1], idx1, isem1).wait()
    pltpu.async_copy(xl.at[idx1.at[0]], rows1, gsem1)
    pltpu.sync_copy(rows0, msg_sh.at[idx0.at[1]], add=True)
    pltpu.async_copy(rc.at[jt + 2], idx0, isem0)
    pltpu.make_async_copy(xl.at[idx1.at[0]], rows1, gsem1).wait()
    pltpu.make_async_copy(rc.at[jt + 2], idx0, isem0).wait()
    pltpu.async_copy(xl.at[idx0.at[0]], rows0, gsem0)
    pltpu.sync_copy(rows1, msg_sh.at[idx1.at[1]], add=True)
    pltpu.make_async_copy(xl.at[idx0.at[0]], rows0, gsem0).wait()
    pltpu.sync_copy(rows0, msg_sh.at[idx0.at[1]], add=True)

    plsc.subcore_barrier()

    pltpu.sync_copy(msg_sh.at[pl.ds(sid * ROWS_PER_TILE, ROWS_PER_TILE)],
                    out.at[cid, pl.ds(sid * ROWS_PER_TILE, ROWS_PER_TILE)])


_sc_aggregate = functools.partial(
    pl.kernel,
    out_type=jax.ShapeDtypeStruct((NC, NPAD, D), jnp.float32),
    mesh=plsc.VectorSubcoreMesh(core_axis_name="c", subcore_axis_name="s",
                                num_cores=NC, num_subcores=NS),
    scratch_types=[
        pltpu.VMEM((2, K), jnp.int32),
        pltpu.VMEM((2, K), jnp.int32),
        pltpu.VMEM((K, D), jnp.float32),
        pltpu.VMEM((K, D), jnp.float32),
        pltpu.VMEM((16, D), jnp.float32),
        pltpu.VMEM_SHARED((NPAD, D), jnp.float32),
        pltpu.SemaphoreType.DMA,
        pltpu.SemaphoreType.DMA,
        pltpu.SemaphoreType.DMA,
        pltpu.SemaphoreType.DMA,
    ],
)(_sc_aggregate_body)


def _lstm_body(msgp_ref, h0_ref, c0_ref, wih_ref, whh_ref, bih_ref, bhh_ref,
               h_ref, c_ref):
    msg = msgp_ref[0] + msgp_ref[1]
    gates = (jnp.dot(msg, wih_ref[...], preferred_element_type=jnp.float32)
             + jnp.dot(h0_ref[...], whh_ref[...],
                       preferred_element_type=jnp.float32)
             + bih_ref[...] + bhh_ref[...])
    i = jax.nn.sigmoid(gates[:, 0 * D:1 * D])
    f = jax.nn.sigmoid(gates[:, 1 * D:2 * D])
    g = jnp.tanh(gates[:, 2 * D:3 * D])
    o = jax.nn.sigmoid(gates[:, 3 * D:4 * D])
    c_new = f * c0_ref[...] + i * g
    c_ref[...] = c_new
    h_ref[...] = o * jnp.tanh(c_new)


def _lstm_call(msgp, h0, c0, wihT, whhT, bih, bhh):
    R = 400
    grid = N // R
    return pl.pallas_call(
        _lstm_body,
        grid=(grid,),
        in_specs=[
            pl.BlockSpec((NC, R, D), lambda i: (0, i, 0)),
            pl.BlockSpec((R, D), lambda i: (i, 0)),
            pl.BlockSpec((R, D), lambda i: (i, 0)),
            pl.BlockSpec((D, 4 * D), lambda i: (0, 0)),
            pl.BlockSpec((D, 4 * D), lambda i: (0, 0)),
            pl.BlockSpec((1, 4 * D), lambda i: (0, 0)),
            pl.BlockSpec((1, 4 * D), lambda i: (0, 0)),
        ],
        out_specs=[
            pl.BlockSpec((R, D), lambda i: (i, 0)),
            pl.BlockSpec((R, D), lambda i: (i, 0)),
        ],
        out_shape=[
            jax.ShapeDtypeStruct((N, D), jnp.float32),
            jax.ShapeDtypeStruct((N, D), jnp.float32),
        ],
    )(msgp, h0, c0, wihT, whhT, bih, bhh)


def kernel(edge_index, x_l, h0, c0, W_ih, W_hh, b_ih, b_hh):
    ei = edge_index.astype(jnp.int32)
    pad = EPAD - E
    rowp = jnp.concatenate([ei[0], jnp.full((pad,), N, jnp.int32)])
    colp = jnp.concatenate([ei[1], jnp.zeros((pad,), jnp.int32)])
    rc = jnp.stack([colp.reshape(NCHUNK, K), rowp.reshape(NCHUNK, K)], axis=1)

    msgp = _sc_aggregate(rc, x_l)

    h_new, c_new = _lstm_call(msgp, h0, c0, W_ih.T, W_hh.T,
                              b_ih.reshape(1, 4 * D), b_hh.reshape(1, 4 * D))
    return (h_new, c_new)


# speedup vs baseline: 4.9248x; 1.0006x over previous
import functools

import jax
import jax.numpy as jnp
from jax import lax
from jax.experimental import pallas as pl
from jax.experimental.pallas import tpu as pltpu
from jax.experimental.pallas import tpu_sc as plsc

D = 128
N = 10000
E = 320000
NC, NS = 2, 16
NW = NC * NS
K = 128
CPW = -(-E // (K * NW))
NCHUNK = CPW * NW
EPAD = NCHUNK * K
ROWS_PER_TILE = 640
NPAD = NS * ROWS_PER_TILE


def _sc_aggregate_body(rc, xl, out, idx0, idx1, rows0, rows1, z16_v,
                       msg_sh, isem0, isem1, gsem0, gsem1):
    cid = lax.axis_index("c")
    sid = lax.axis_index("s")
    wid = cid * NS + sid
    c0 = wid * CPW

    zero = jnp.zeros((16,), jnp.float32)
    for i in range(16):
        for j in range(D // 16):
            z16_v[i, pl.ds(j * 16, 16)] = zero

    def zero_body(j, carry):
        pltpu.sync_copy(z16_v, msg_sh.at[pl.ds(sid * ROWS_PER_TILE + j * 16, 16)])
        return carry
    lax.fori_loop(0, ROWS_PER_TILE // 16, zero_body, 0)
    plsc.subcore_barrier()

    pltpu.async_copy(rc.at[c0], idx0, isem0).wait()
    pltpu.async_copy(xl.at[idx0.at[0]], rows0, gsem0)
    pltpu.async_copy(rc.at[c0 + 1], idx1, isem1)

    def acc_body(t, carry):
        j = c0 + 2 * t
        pltpu.make_async_copy(xl.at[idx0.at[0]], rows0, gsem0).wait()
        pltpu.make_async_copy(rc.at[j + 1], idx1, isem1).wait()
        pltpu.async_copy(xl.at[idx1.at[0]], rows1, gsem1)
        pltpu.sync_copy(rows0, msg_sh.at[idx0.at[1]], add=True)
        pltpu.async_copy(rc.at[j + 2], idx0, isem0)
        pltpu.make_async_copy(xl.at[idx1.at[0]], rows1, gsem1).wait()
        pltpu.make_async_copy(rc.at[j + 2], idx0, isem0).wait()
        pltpu.async_copy(xl.at[idx0.at[0]], rows0, gsem0)
        pltpu.sync_copy(rows1, msg_sh.at[idx1.at[1]], add=True)
        pltpu.async_copy(rc.at[j + 3], idx1, isem1)
        return carry
    lax.fori_loop(0, (CPW - 3) // 2, acc_body, 0)

    jt = c0 + CPW - 3
    pltpu.make_async_copy(xl.at[idx0.at[0]], rows0, gsem0).wait()
    pltpu.make_async_copy(rc.at[jt + 1], idx1, isem1).wait()
    pltpu.async_copy(xl.at[idx1.at[0]], rows1, gsem1)
    pltpu.sync_copy(rows0, msg_sh.at[idx0.at[1]], add=True)
    pltpu.async_copy(rc.at[jt + 2], idx0, isem0)
    pltpu.make_async_copy(xl.at[idx1.at[0]], rows1, gsem1).wait()
    pltpu.make_async_copy(rc.at[jt + 2], idx0, isem0).wait()
    pltpu.async_copy(xl.at[idx0.at[0]], rows0, gsem0)
    pltpu.sync_copy(rows1, msg_sh.at[idx1.at[1]], add=True)
    pltpu.make_async_copy(xl.at[idx0.at[0]], rows0, gsem0).wait()
    pltpu.sync_copy(rows0, msg_sh.at[idx0.at[1]], add=True)

    plsc.subcore_barrier()

    pltpu.sync_copy(msg_sh.at[pl.ds(sid * ROWS_PER_TILE, ROWS_PER_TILE)],
                    out.at[cid, pl.ds(sid * ROWS_PER_TILE, ROWS_PER_TILE)])


_sc_aggregate = functools.partial(
    pl.kernel,
    out_type=jax.ShapeDtypeStruct((NC, NPAD, D), jnp.float32),
    mesh=plsc.VectorSubcoreMesh(core_axis_name="c", subcore_axis_name="s",
                                num_cores=NC, num_subcores=NS),
    scratch_types=[
        pltpu.VMEM((2, K), jnp.int32),
        pltpu.VMEM((2, K), jnp.int32),
        pltpu.VMEM((K, D), jnp.float32),
        pltpu.VMEM((K, D), jnp.float32),
        pltpu.VMEM((16, D), jnp.float32),
        pltpu.VMEM_SHARED((NPAD, D), jnp.float32),
        pltpu.SemaphoreType.DMA,
        pltpu.SemaphoreType.DMA,
        pltpu.SemaphoreType.DMA,
        pltpu.SemaphoreType.DMA,
    ],
)(_sc_aggregate_body)


def _lstm_body(msgp_ref, h0_ref, c0_ref, wih_ref, whh_ref, bih_ref, bhh_ref,
               h_ref, c_ref):
    msg = msgp_ref[0] + msgp_ref[1]
    gates = (jnp.dot(msg, wih_ref[...], preferred_element_type=jnp.float32)
             + jnp.dot(h0_ref[...], whh_ref[...],
                       preferred_element_type=jnp.float32)
             + bih_ref[...] + bhh_ref[...])
    i = jax.nn.sigmoid(gates[:, 0 * D:1 * D])
    f = jax.nn.sigmoid(gates[:, 1 * D:2 * D])
    g = jnp.tanh(gates[:, 2 * D:3 * D])
    o = jax.nn.sigmoid(gates[:, 3 * D:4 * D])
    c_new = f * c0_ref[...] + i * g
    c_ref[...] = c_new
    h_ref[...] = o * jnp.tanh(c_new)


def _lstm_call(msgp, h0, c0, wihT, whhT, bih, bhh):
    R = 400
    grid = N // R
    return pl.pallas_call(
        _lstm_body,
        grid=(grid,),
        in_specs=[
            pl.BlockSpec((NC, R, D), lambda i: (0, i, 0)),
            pl.BlockSpec((R, D), lambda i: (i, 0)),
            pl.BlockSpec((R, D), lambda i: (i, 0)),
            pl.BlockSpec((D, 4 * D), lambda i: (0, 0)),
            pl.BlockSpec((D, 4 * D), lambda i: (0, 0)),
            pl.BlockSpec((1, 4 * D), lambda i: (0, 0)),
            pl.BlockSpec((1, 4 * D), lambda i: (0, 0)),
        ],
        out_specs=[
            pl.BlockSpec((R, D), lambda i: (i, 0)),
            pl.BlockSpec((R, D), lambda i: (i, 0)),
        ],
        out_shape=[
            jax.ShapeDtypeStruct((N, D), jnp.float32),
            jax.ShapeDtypeStruct((N, D), jnp.float32),
        ],
    )(msgp, h0, c0, wihT, whhT, bih, bhh)


def kernel(edge_index, x_l, h0, c0, W_ih, W_hh, b_ih, b_hh):
    ei = edge_index.astype(jnp.int32)
    pad = EPAD - E
    dummy_rows = N + jnp.arange(pad, dtype=jnp.int32) % (NPAD - N)
    rowp = jnp.concatenate([ei[0], dummy_rows])
    colp = jnp.concatenate([ei[1], jnp.zeros((pad,), jnp.int32)])
    rc = jnp.stack([colp.reshape(NCHUNK, K), rowp.reshape(NCHUNK, K)], axis=1)

    msgp = _sc_aggregate(rc, x_l)

    h_new, c_new = _lstm_call(msgp, h0, c0, W_ih.T, W_hh.T,
                              b_ih.reshape(1, 4 * D), b_hh.reshape(1, 4 * D))
    return (h_new, c_new)


# speedup vs baseline: 4.9308x; 1.0012x over previous
import functools

import jax
import jax.numpy as jnp
from jax import lax
from jax.experimental import pallas as pl
from jax.experimental.pallas import tpu as pltpu
from jax.experimental.pallas import tpu_sc as plsc

D = 128
N = 10000
E = 320000
NC, NS = 2, 16
NW = NC * NS
K = 128
CPW = -(-E // (K * NW))
NCHUNK = CPW * NW
EPAD = NCHUNK * K
ROWS_PER_TILE = 640
NPAD = NS * ROWS_PER_TILE


def _sc_aggregate_body(rc, xl, out, idx0, idx1, rows0, rows1, z16_v,
                       msg_sh, isem0, isem1, gsem0, gsem1):
    cid = lax.axis_index("c")
    sid = lax.axis_index("s")
    wid = cid * NS + sid
    c0 = wid * CPW

    with jax.named_scope("zero_phase"):
        zero = jnp.zeros((16,), jnp.float32)
        for i in range(16):
            for j in range(D // 16):
                z16_v[i, pl.ds(j * 16, 16)] = zero

        def zero_body(j, carry):
            pltpu.sync_copy(z16_v, msg_sh.at[pl.ds(sid * ROWS_PER_TILE + j * 16, 16)])
            return carry
        lax.fori_loop(0, ROWS_PER_TILE // 16, zero_body, 0)
        plsc.subcore_barrier()

    scope_acc = jax.named_scope("acc_phase")
    scope_acc.__enter__()
    pltpu.async_copy(rc.at[c0], idx0, isem0).wait()
    pltpu.async_copy(xl.at[idx0.at[0]], rows0, gsem0)
    pltpu.async_copy(rc.at[c0 + 1], idx1, isem1)

    def acc_body(t, carry):
        j = c0 + 2 * t
        pltpu.make_async_copy(xl.at[idx0.at[0]], rows0, gsem0).wait()
        pltpu.make_async_copy(rc.at[j + 1], idx1, isem1).wait()
        pltpu.async_copy(xl.at[idx1.at[0]], rows1, gsem1)
        pltpu.sync_copy(rows0, msg_sh.at[idx0.at[1]], add=True)
        pltpu.async_copy(rc.at[j + 2], idx0, isem0)
        pltpu.make_async_copy(xl.at[idx1.at[0]], rows1, gsem1).wait()
        pltpu.make_async_copy(rc.at[j + 2], idx0, isem0).wait()
        pltpu.async_copy(xl.at[idx0.at[0]], rows0, gsem0)
        pltpu.sync_copy(rows1, msg_sh.at[idx1.at[1]], add=True)
        pltpu.async_copy(rc.at[j + 3], idx1, isem1)
        return carry
    lax.fori_loop(0, (CPW - 3) // 2, acc_body, 0)

    jt = c0 + CPW - 3
    pltpu.make_async_copy(xl.at[idx0.at[0]], rows0, gsem0).wait()
    pltpu.make_async_copy(rc.at[jt + 1], idx1, isem1).wait()
    pltpu.async_copy(xl.at[idx1.at[0]], rows1, gsem1)
    pltpu.sync_copy(rows0, msg_sh.at[idx0.at[1]], add=True)
    pltpu.async_copy(rc.at[jt + 2], idx0, isem0)
    pltpu.make_async_copy(xl.at[idx1.at[0]], rows1, gsem1).wait()
    pltpu.make_async_copy(rc.at[jt + 2], idx0, isem0).wait()
    pltpu.async_copy(xl.at[idx0.at[0]], rows0, gsem0)
    pltpu.sync_copy(rows1, msg_sh.at[idx1.at[1]], add=True)
    pltpu.make_async_copy(xl.at[idx0.at[0]], rows0, gsem0).wait()
    pltpu.sync_copy(rows0, msg_sh.at[idx0.at[1]], add=True)

    plsc.subcore_barrier()
    scope_acc.__exit__(None, None, None)

    with jax.named_scope("out_phase"):
        pltpu.sync_copy(msg_sh.at[pl.ds(sid * ROWS_PER_TILE, ROWS_PER_TILE)],
                        out.at[cid, pl.ds(sid * ROWS_PER_TILE, ROWS_PER_TILE)])


_sc_aggregate = functools.partial(
    pl.kernel,
    out_type=jax.ShapeDtypeStruct((NC, NPAD, D), jnp.float32),
    mesh=plsc.VectorSubcoreMesh(core_axis_name="c", subcore_axis_name="s",
                                num_cores=NC, num_subcores=NS),
    scratch_types=[
        pltpu.VMEM((2, K), jnp.int32),
        pltpu.VMEM((2, K), jnp.int32),
        pltpu.VMEM((K, D), jnp.float32),
        pltpu.VMEM((K, D), jnp.float32),
        pltpu.VMEM((16, D), jnp.float32),
        pltpu.VMEM_SHARED((NPAD, D), jnp.float32),
        pltpu.SemaphoreType.DMA,
        pltpu.SemaphoreType.DMA,
        pltpu.SemaphoreType.DMA,
        pltpu.SemaphoreType.DMA,
    ],
)(_sc_aggregate_body)


def _lstm_body(msgp_ref, h0_ref, c0_ref, wih_ref, whh_ref, bih_ref, bhh_ref,
               h_ref, c_ref):
    msg = msgp_ref[0] + msgp_ref[1]
    gates = (jnp.dot(msg, wih_ref[...], preferred_element_type=jnp.float32)
             + jnp.dot(h0_ref[...], whh_ref[...],
                       preferred_element_type=jnp.float32)
             + bih_ref[...] + bhh_ref[...])
    i = jax.nn.sigmoid(gates[:, 0 * D:1 * D])
    f = jax.nn.sigmoid(gates[:, 1 * D:2 * D])
    g = jnp.tanh(gates[:, 2 * D:3 * D])
    o = jax.nn.sigmoid(gates[:, 3 * D:4 * D])
    c_new = f * c0_ref[...] + i * g
    c_ref[...] = c_new
    h_ref[...] = o * jnp.tanh(c_new)


def _lstm_call(msgp, h0, c0, wihT, whhT, bih, bhh):
    R = 400
    grid = N // R
    return pl.pallas_call(
        _lstm_body,
        grid=(grid,),
        in_specs=[
            pl.BlockSpec((NC, R, D), lambda i: (0, i, 0)),
            pl.BlockSpec((R, D), lambda i: (i, 0)),
            pl.BlockSpec((R, D), lambda i: (i, 0)),
            pl.BlockSpec((D, 4 * D), lambda i: (0, 0)),
            pl.BlockSpec((D, 4 * D), lambda i: (0, 0)),
            pl.BlockSpec((1, 4 * D), lambda i: (0, 0)),
            pl.BlockSpec((1, 4 * D), lambda i: (0, 0)),
        ],
        out_specs=[
            pl.BlockSpec((R, D), lambda i: (i, 0)),
            pl.BlockSpec((R, D), lambda i: (i, 0)),
        ],
        out_shape=[
            jax.ShapeDtypeStruct((N, D), jnp.float32),
            jax.ShapeDtypeStruct((N, D), jnp.float32),
        ],
    )(msgp, h0, c0, wihT, whhT, bih, bhh)


def kernel(edge_index, x_l, h0, c0, W_ih, W_hh, b_ih, b_hh):
    ei = edge_index.astype(jnp.int32)
    pad = EPAD - E
    dummy_rows = N + jnp.arange(pad, dtype=jnp.int32) % (NPAD - N)
    rowp = jnp.concatenate([ei[0], dummy_rows])
    colp = jnp.concatenate([ei[1], jnp.zeros((pad,), jnp.int32)])
    rc = jnp.stack([colp.reshape(NCHUNK, K), rowp.reshape(NCHUNK, K)], axis=1)

    msgp = _sc_aggregate(rc, x_l)

    h_new, c_new = _lstm_call(msgp, h0, c0, W_ih.T, W_hh.T,
                              b_ih.reshape(1, 4 * D), b_hh.reshape(1, 4 * D))
    return (h_new, c_new)


# speedup vs baseline: 5.9144x; 1.1995x over previous
import functools

import jax
import jax.numpy as jnp
from jax import lax
from jax.experimental import pallas as pl
from jax.experimental.pallas import tpu as pltpu
from jax.experimental.pallas import tpu_sc as plsc

D = 128
N = 10000
E = 320000
NC, NS = 2, 16
NW = NC * NS
K = 112
NBUF = 3
CPW = 90
NCHUNK = CPW * NW
EPAD = NCHUNK * K
ROWS_PER_TILE = 640
NPAD = NS * ROWS_PER_TILE


def _sc_aggregate_body(rc, xl, out, idx, rows, z16_v, msg_sh, isem, gsem):
    cid = lax.axis_index("c")
    sid = lax.axis_index("s")
    wid = cid * NS + sid
    c0 = wid * CPW

    zero = jnp.zeros((16,), jnp.float32)
    for i in range(16):
        for j in range(D // 16):
            z16_v[i, pl.ds(j * 16, 16)] = zero

    def zero_body(j, carry):
        pltpu.sync_copy(z16_v, msg_sh.at[pl.ds(sid * ROWS_PER_TILE + j * 16, 16)])
        return carry
    lax.fori_loop(0, ROWS_PER_TILE // 16, zero_body, 0)
    plsc.subcore_barrier()

    def seg(j, b, b2):
        pltpu.make_async_copy(rc.at[j + 2], idx[b2], isem[b2]).wait()
        pltpu.async_copy(xl.at[idx[b2].at[0]], rows[b2], gsem[b2])
        pltpu.make_async_copy(xl.at[idx[b].at[0]], rows[b], gsem[b]).wait()
        pltpu.sync_copy(rows[b], msg_sh.at[idx[b].at[1]], add=True)
        pltpu.async_copy(rc.at[j + 3], idx[b], isem[b])

    for b in range(2):
        pltpu.async_copy(rc.at[c0 + b], idx[b], isem[b])
    for b in range(2):
        pltpu.make_async_copy(rc.at[c0 + b], idx[b], isem[b]).wait()
        pltpu.async_copy(xl.at[idx[b].at[0]], rows[b], gsem[b])
    pltpu.async_copy(rc.at[c0 + 2], idx[2], isem[2])

    def acc_body(t, carry):
        j = c0 + 3 * t
        for b in range(3):
            seg(j + b, b, (b + 2) % 3)
        return carry
    lax.fori_loop(0, (CPW - 3) // 3, acc_body, 0)

    pltpu.make_async_copy(rc.at[c0 + CPW - 1], idx[2], isem[2]).wait()
    pltpu.async_copy(xl.at[idx[2].at[0]], rows[2], gsem[2])
    for b in range(3):
        pltpu.make_async_copy(xl.at[idx[b].at[0]], rows[b], gsem[b]).wait()
        pltpu.sync_copy(rows[b], msg_sh.at[idx[b].at[1]], add=True)

    plsc.subcore_barrier()

    pltpu.sync_copy(msg_sh.at[pl.ds(sid * ROWS_PER_TILE, ROWS_PER_TILE)],
                    out.at[cid, pl.ds(sid * ROWS_PER_TILE, ROWS_PER_TILE)])


_sc_aggregate = functools.partial(
    pl.kernel,
    out_type=jax.ShapeDtypeStruct((NC, NPAD, D), jnp.float32),
    mesh=plsc.VectorSubcoreMesh(core_axis_name="c", subcore_axis_name="s",
                                num_cores=NC, num_subcores=NS),
    scratch_types=[
        [pltpu.VMEM((2, K), jnp.int32) for _ in range(NBUF)],
        [pltpu.VMEM((K, D), jnp.float32) for _ in range(NBUF)],
        pltpu.VMEM((16, D), jnp.float32),
        pltpu.VMEM_SHARED((NPAD, D), jnp.float32),
        [pltpu.SemaphoreType.DMA for _ in range(NBUF)],
        [pltpu.SemaphoreType.DMA for _ in range(NBUF)],
    ],
)(_sc_aggregate_body)


def _lstm_body(msgp_ref, h0_ref, c0_ref, wih_ref, whh_ref, bih_ref, bhh_ref,
               h_ref, c_ref):
    msg = msgp_ref[0] + msgp_ref[1]
    gates = (jnp.dot(msg, wih_ref[...], preferred_element_type=jnp.float32)
             + jnp.dot(h0_ref[...], whh_ref[...],
                       preferred_element_type=jnp.float32)
             + bih_ref[...] + bhh_ref[...])
    i = jax.nn.sigmoid(gates[:, 0 * D:1 * D])
    f = jax.nn.sigmoid(gates[:, 1 * D:2 * D])
    g = jnp.tanh(gates[:, 2 * D:3 * D])
    o = jax.nn.sigmoid(gates[:, 3 * D:4 * D])
    c_new = f * c0_ref[...] + i * g
    c_ref[...] = c_new
    h_ref[...] = o * jnp.tanh(c_new)


def _lstm_call(msgp, h0, c0, wihT, whhT, bih, bhh):
    R = 400
    grid = N // R
    return pl.pallas_call(
        _lstm_body,
        grid=(grid,),
        in_specs=[
            pl.BlockSpec((NC, R, D), lambda i: (0, i, 0)),
            pl.BlockSpec((R, D), lambda i: (i, 0)),
            pl.BlockSpec((R, D), lambda i: (i, 0)),
            pl.BlockSpec((D, 4 * D), lambda i: (0, 0)),
            pl.BlockSpec((D, 4 * D), lambda i: (0, 0)),
            pl.BlockSpec((1, 4 * D), lambda i: (0, 0)),
            pl.BlockSpec((1, 4 * D), lambda i: (0, 0)),
        ],
        out_specs=[
            pl.BlockSpec((R, D), lambda i: (i, 0)),
            pl.BlockSpec((R, D), lambda i: (i, 0)),
        ],
        out_shape=[
            jax.ShapeDtypeStruct((N, D), jnp.float32),
            jax.ShapeDtypeStruct((N, D), jnp.float32),
        ],
    )(msgp, h0, c0, wihT, whhT, bih, bhh)


def kernel(edge_index, x_l, h0, c0, W_ih, W_hh, b_ih, b_hh):
    ei = edge_index.astype(jnp.int32)
    pad = EPAD - E
    dummy_rows = N + jnp.arange(pad, dtype=jnp.int32) % (NPAD - N)
    rowp = jnp.concatenate([ei[0], dummy_rows])
    colp = jnp.concatenate([ei[1], jnp.zeros((pad,), jnp.int32)])
    rc = jnp.stack([colp.reshape(NCHUNK, K), rowp.reshape(NCHUNK, K)], axis=1)

    msgp = _sc_aggregate(rc, x_l)

    h_new, c_new = _lstm_call(msgp, h0, c0, W_ih.T, W_hh.T,
                              b_ih.reshape(1, 4 * D), b_hh.reshape(1, 4 * D))
    return (h_new, c_new)


# speedup vs baseline: 6.6685x; 1.1275x over previous
import functools

import jax
import jax.numpy as jnp
from jax import lax
from jax.experimental import pallas as pl
from jax.experimental.pallas import tpu as pltpu
from jax.experimental.pallas import tpu_sc as plsc

D = 128
N = 10000
E = 320000
NC, NS = 2, 16
NW = NC * NS
K = 112
NBUF = 3
CPW_A = 120
CPW_B = 60
NCHUNK = NS * (CPW_A + CPW_B)
EPAD = NCHUNK * K
ROWS_PER_TILE = 640
NPAD = NS * ROWS_PER_TILE


def _sc_aggregate_body(rc, xl, out, idx, rows, z16_v, msg_sh, isem, gsem):
    cid = lax.axis_index("c")
    sid = lax.axis_index("s")
    cpw = jnp.where(cid == 0, CPW_A, CPW_B)
    c0 = cid * NS * CPW_A + sid * cpw
    steady_trip = jnp.where(cid == 0, (CPW_A - 3) // 3, (CPW_B - 3) // 3)

    zero = jnp.zeros((16,), jnp.float32)
    for i in range(16):
        for j in range(D // 16):
            z16_v[i, pl.ds(j * 16, 16)] = zero

    def zero_body(j, carry):
        pltpu.sync_copy(z16_v, msg_sh.at[pl.ds(sid * ROWS_PER_TILE + j * 16, 16)])
        return carry
    lax.fori_loop(0, ROWS_PER_TILE // 16, zero_body, 0)
    plsc.subcore_barrier()

    def seg(j, b, b2):
        pltpu.make_async_copy(rc.at[j + 2], idx[b2], isem[b2]).wait()
        pltpu.async_copy(xl.at[idx[b2].at[0]], rows[b2], gsem[b2])
        pltpu.make_async_copy(xl.at[idx[b].at[0]], rows[b], gsem[b]).wait()
        pltpu.sync_copy(rows[b], msg_sh.at[idx[b].at[1]], add=True)
        pltpu.async_copy(rc.at[j + 3], idx[b], isem[b])

    for b in range(2):
        pltpu.async_copy(rc.at[c0 + b], idx[b], isem[b])
    for b in range(2):
        pltpu.make_async_copy(rc.at[c0 + b], idx[b], isem[b]).wait()
        pltpu.async_copy(xl.at[idx[b].at[0]], rows[b], gsem[b])
    pltpu.async_copy(rc.at[c0 + 2], idx[2], isem[2])

    def acc_body(t, carry):
        j = c0 + 3 * t
        for b in range(3):
            seg(j + b, b, (b + 2) % 3)
        return carry
    lax.fori_loop(0, steady_trip, acc_body, 0)

    pltpu.make_async_copy(rc.at[c0 + cpw - 1], idx[2], isem[2]).wait()
    pltpu.async_copy(xl.at[idx[2].at[0]], rows[2], gsem[2])
    for b in range(3):
        pltpu.make_async_copy(xl.at[idx[b].at[0]], rows[b], gsem[b]).wait()
        pltpu.sync_copy(rows[b], msg_sh.at[idx[b].at[1]], add=True)

    plsc.subcore_barrier()

    pltpu.sync_copy(msg_sh.at[pl.ds(sid * ROWS_PER_TILE, ROWS_PER_TILE)],
                    out.at[cid, pl.ds(sid * ROWS_PER_TILE, ROWS_PER_TILE)])


_sc_aggregate = functools.partial(
    pl.kernel,
    out_type=jax.ShapeDtypeStruct((NC, NPAD, D), jnp.float32),
    mesh=plsc.VectorSubcoreMesh(core_axis_name="c", subcore_axis_name="s",
                                num_cores=NC, num_subcores=NS),
    scratch_types=[
        [pltpu.VMEM((2, K), jnp.int32) for _ in range(NBUF)],
        [pltpu.VMEM((K, D), jnp.float32) for _ in range(NBUF)],
        pltpu.VMEM((16, D), jnp.float32),
        pltpu.VMEM_SHARED((NPAD, D), jnp.float32),
        [pltpu.SemaphoreType.DMA for _ in range(NBUF)],
        [pltpu.SemaphoreType.DMA for _ in range(NBUF)],
    ],
)(_sc_aggregate_body)


def _lstm_body(msgp_ref, h0_ref, c0_ref, wih_ref, whh_ref, bih_ref, bhh_ref,
               h_ref, c_ref):
    msg = msgp_ref[0] + msgp_ref[1]
    gates = (jnp.dot(msg, wih_ref[...], preferred_element_type=jnp.float32)
             + jnp.dot(h0_ref[...], whh_ref[...],
                       preferred_element_type=jnp.float32)
             + bih_ref[...] + bhh_ref[...])
    i = jax.nn.sigmoid(gates[:, 0 * D:1 * D])
    f = jax.nn.sigmoid(gates[:, 1 * D:2 * D])
    g = jnp.tanh(gates[:, 2 * D:3 * D])
    o = jax.nn.sigmoid(gates[:, 3 * D:4 * D])
    c_new = f * c0_ref[...] + i * g
    c_ref[...] = c_new
    h_ref[...] = o * jnp.tanh(c_new)


def _lstm_call(msgp, h0, c0, wihT, whhT, bih, bhh):
    R = 400
    grid = N // R
    return pl.pallas_call(
        _lstm_body,
        grid=(grid,),
        in_specs=[
            pl.BlockSpec((NC, R, D), lambda i: (0, i, 0)),
            pl.BlockSpec((R, D), lambda i: (i, 0)),
            pl.BlockSpec((R, D), lambda i: (i, 0)),
            pl.BlockSpec((D, 4 * D), lambda i: (0, 0)),
            pl.BlockSpec((D, 4 * D), lambda i: (0, 0)),
            pl.BlockSpec((1, 4 * D), lambda i: (0, 0)),
            pl.BlockSpec((1, 4 * D), lambda i: (0, 0)),
        ],
        out_specs=[
            pl.BlockSpec((R, D), lambda i: (i, 0)),
            pl.BlockSpec((R, D), lambda i: (i, 0)),
        ],
        out_shape=[
            jax.ShapeDtypeStruct((N, D), jnp.float32),
            jax.ShapeDtypeStruct((N, D), jnp.float32),
        ],
    )(msgp, h0, c0, wihT, whhT, bih, bhh)


def kernel(edge_index, x_l, h0, c0, W_ih, W_hh, b_ih, b_hh):
    ei = edge_index.astype(jnp.int32)
    pad = EPAD - E
    dummy_rows = N + jnp.arange(pad, dtype=jnp.int32) % (NPAD - N)
    rowp = jnp.concatenate([ei[0], dummy_rows])
    colp = jnp.concatenate([ei[1], jnp.zeros((pad,), jnp.int32)])
    rc = jnp.stack([colp.reshape(NCHUNK, K), rowp.reshape(NCHUNK, K)], axis=1)

    msgp = _sc_aggregate(rc, x_l)

    h_new, c_new = _lstm_call(msgp, h0, c0, W_ih.T, W_hh.T,
                              b_ih.reshape(1, 4 * D), b_hh.reshape(1, 4 * D))
    return (h_new, c_new)


# speedup vs baseline: 7.2822x; 1.0920x over previous
import functools

import jax
import jax.numpy as jnp
from jax import lax
from jax.experimental import pallas as pl
from jax.experimental.pallas import tpu as pltpu
from jax.experimental.pallas import tpu_sc as plsc

D = 128
N = 10000
E = 320000
NC, NS = 2, 16
NW = NC * NS
K = 112
NBUF = 3
CPW_A = 141
CPW_B = 39
NCHUNK = NS * (CPW_A + CPW_B)
EPAD = NCHUNK * K
ROWS_PER_TILE = 640
NPAD = NS * ROWS_PER_TILE


def _sc_aggregate_body(rc, xl, out, idx, rows, z16_v, msg_sh, isem, gsem):
    cid = lax.axis_index("c")
    sid = lax.axis_index("s")
    cpw = jnp.where(cid == 0, CPW_A, CPW_B)
    c0 = cid * NS * CPW_A + sid * cpw
    steady_trip = jnp.where(cid == 0, (CPW_A - 3) // 3, (CPW_B - 3) // 3)

    zero = jnp.zeros((16,), jnp.float32)
    for i in range(16):
        for j in range(D // 16):
            z16_v[i, pl.ds(j * 16, 16)] = zero

    def zero_body(j, carry):
        pltpu.sync_copy(z16_v, msg_sh.at[pl.ds(sid * ROWS_PER_TILE + j * 16, 16)])
        return carry
    lax.fori_loop(0, ROWS_PER_TILE // 16, zero_body, 0)
    plsc.subcore_barrier()

    def seg(j, b, b2):
        pltpu.make_async_copy(rc.at[j + 2], idx[b2], isem[b2]).wait()
        pltpu.async_copy(xl.at[idx[b2].at[0]], rows[b2], gsem[b2])
        pltpu.make_async_copy(xl.at[idx[b].at[0]], rows[b], gsem[b]).wait()
        pltpu.sync_copy(rows[b], msg_sh.at[idx[b].at[1]], add=True)
        pltpu.async_copy(rc.at[j + 3], idx[b], isem[b])

    for b in range(2):
        pltpu.async_copy(rc.at[c0 + b], idx[b], isem[b])
    for b in range(2):
        pltpu.make_async_copy(rc.at[c0 + b], idx[b], isem[b]).wait()
        pltpu.async_copy(xl.at[idx[b].at[0]], rows[b], gsem[b])
    pltpu.async_copy(rc.at[c0 + 2], idx[2], isem[2])

    def acc_body(t, carry):
        j = c0 + 3 * t
        for b in range(3):
            seg(j + b, b, (b + 2) % 3)
        return carry
    lax.fori_loop(0, steady_trip, acc_body, 0)

    pltpu.make_async_copy(rc.at[c0 + cpw - 1], idx[2], isem[2]).wait()
    pltpu.async_copy(xl.at[idx[2].at[0]], rows[2], gsem[2])
    for b in range(3):
        pltpu.make_async_copy(xl.at[idx[b].at[0]], rows[b], gsem[b]).wait()
        pltpu.sync_copy(rows[b], msg_sh.at[idx[b].at[1]], add=True)

    plsc.subcore_barrier()

    pltpu.sync_copy(msg_sh.at[pl.ds(sid * ROWS_PER_TILE, ROWS_PER_TILE)],
                    out.at[cid, pl.ds(sid * ROWS_PER_TILE, ROWS_PER_TILE)])


_sc_aggregate = functools.partial(
    pl.kernel,
    out_type=jax.ShapeDtypeStruct((NC, NPAD, D), jnp.float32),
    mesh=plsc.VectorSubcoreMesh(core_axis_name="c", subcore_axis_name="s",
                                num_cores=NC, num_subcores=NS),
    scratch_types=[
        [pltpu.VMEM((2, K), jnp.int32) for _ in range(NBUF)],
        [pltpu.VMEM((K, D), jnp.float32) for _ in range(NBUF)],
        pltpu.VMEM((16, D), jnp.float32),
        pltpu.VMEM_SHARED((NPAD, D), jnp.float32),
        [pltpu.SemaphoreType.DMA for _ in range(NBUF)],
        [pltpu.SemaphoreType.DMA for _ in range(NBUF)],
    ],
)(_sc_aggregate_body)


def _lstm_body(msgp_ref, h0_ref, c0_ref, wih_ref, whh_ref, bih_ref, bhh_ref,
               h_ref, c_ref):
    msg = msgp_ref[0] + msgp_ref[1]
    gates = (jnp.dot(msg, wih_ref[...], preferred_element_type=jnp.float32)
             + jnp.dot(h0_ref[...], whh_ref[...],
                       preferred_element_type=jnp.float32)
             + bih_ref[...] + bhh_ref[...])
    i = jax.nn.sigmoid(gates[:, 0 * D:1 * D])
    f = jax.nn.sigmoid(gates[:, 1 * D:2 * D])
    g = jnp.tanh(gates[:, 2 * D:3 * D])
    o = jax.nn.sigmoid(gates[:, 3 * D:4 * D])
    c_new = f * c0_ref[...] + i * g
    c_ref[...] = c_new
    h_ref[...] = o * jnp.tanh(c_new)


def _lstm_call(msgp, h0, c0, wihT, whhT, bih, bhh):
    R = 400
    grid = N // R
    return pl.pallas_call(
        _lstm_body,
        grid=(grid,),
        in_specs=[
            pl.BlockSpec((NC, R, D), lambda i: (0, i, 0)),
            pl.BlockSpec((R, D), lambda i: (i, 0)),
            pl.BlockSpec((R, D), lambda i: (i, 0)),
            pl.BlockSpec((D, 4 * D), lambda i: (0, 0)),
            pl.BlockSpec((D, 4 * D), lambda i: (0, 0)),
            pl.BlockSpec((1, 4 * D), lambda i: (0, 0)),
            pl.BlockSpec((1, 4 * D), lambda i: (0, 0)),
        ],
        out_specs=[
            pl.BlockSpec((R, D), lambda i: (i, 0)),
            pl.BlockSpec((R, D), lambda i: (i, 0)),
        ],
        out_shape=[
            jax.ShapeDtypeStruct((N, D), jnp.float32),
            jax.ShapeDtypeStruct((N, D), jnp.float32),
        ],
    )(msgp, h0, c0, wihT, whhT, bih, bhh)


def kernel(edge_index, x_l, h0, c0, W_ih, W_hh, b_ih, b_hh):
    ei = edge_index.astype(jnp.int32)
    pad = EPAD - E
    dummy_rows = N + jnp.arange(pad, dtype=jnp.int32) % (NPAD - N)
    rowp = jnp.concatenate([ei[0], dummy_rows])
    colp = jnp.concatenate([ei[1], jnp.zeros((pad,), jnp.int32)])
    rc = jnp.stack([colp.reshape(NCHUNK, K), rowp.reshape(NCHUNK, K)], axis=1)

    msgp = _sc_aggregate(rc, x_l)

    h_new, c_new = _lstm_call(msgp, h0, c0, W_ih.T, W_hh.T,
                              b_ih.reshape(1, 4 * D), b_hh.reshape(1, 4 * D))
    return (h_new, c_new)


# speedup vs baseline: 7.6088x; 1.0448x over previous
import functools

import jax
import jax.numpy as jnp
from jax import lax
from jax.experimental import pallas as pl
from jax.experimental.pallas import tpu as pltpu
from jax.experimental.pallas import tpu_sc as plsc

D = 128
N = 10000
E = 320000
NC, NS = 2, 16
NW = NC * NS
K = 112
NBUF = 3
CPW_A = 141
CPW_B = 39
NCHUNK = NS * (CPW_A + CPW_B)
EPAD = NCHUNK * K
ROWS_PER_TILE = 640
NPAD = NS * ROWS_PER_TILE


def _sc_aggregate_body(rowp, colp, xl, out, cidx, ridx, rows, z16_v, msg_sh,
                       isem, gsem):
    cid = lax.axis_index("c")
    sid = lax.axis_index("s")
    cpw = jnp.where(cid == 0, CPW_A, CPW_B)
    c0 = cid * NS * CPW_A + sid * cpw
    steady_trip = jnp.where(cid == 0, (CPW_A - 3) // 3, (CPW_B - 3) // 3)

    zero = jnp.zeros((16,), jnp.float32)
    for i in range(16):
        for j in range(D // 16):
            z16_v[i, pl.ds(j * 16, 16)] = zero

    def zero_body(j, carry):
        pltpu.sync_copy(z16_v, msg_sh.at[pl.ds(sid * ROWS_PER_TILE + j * 16, 16)])
        return carry
    lax.fori_loop(0, ROWS_PER_TILE // 16, zero_body, 0)
    plsc.subcore_barrier()

    def load_idx(j, b):
        pltpu.async_copy(colp.at[pl.ds(j * K, K)], cidx[b], isem[b])
        pltpu.async_copy(rowp.at[pl.ds(j * K, K)], ridx[b], isem[b])

    def wait_idx(j, b):
        pltpu.make_async_copy(colp.at[pl.ds(j * K, K)], cidx[b], isem[b]).wait()
        pltpu.make_async_copy(rowp.at[pl.ds(j * K, K)], ridx[b], isem[b]).wait()

    def seg(j, b, b2):
        wait_idx(j + 2, b2)
        pltpu.async_copy(xl.at[cidx[b2]], rows[b2], gsem[b2])
        pltpu.make_async_copy(xl.at[cidx[b]], rows[b], gsem[b]).wait()
        pltpu.sync_copy(rows[b], msg_sh.at[ridx[b]], add=True)
        load_idx(j + 3, b)

    for b in range(2):
        load_idx(c0 + b, b)
    for b in range(2):
        wait_idx(c0 + b, b)
        pltpu.async_copy(xl.at[cidx[b]], rows[b], gsem[b])
    load_idx(c0 + 2, 2)

    def acc_body(t, carry):
        j = c0 + 3 * t
        for b in range(3):
            seg(j + b, b, (b + 2) % 3)
        return carry
    lax.fori_loop(0, steady_trip, acc_body, 0)

    wait_idx(c0 + cpw - 1, 2)
    pltpu.async_copy(xl.at[cidx[2]], rows[2], gsem[2])
    for b in range(3):
        pltpu.make_async_copy(xl.at[cidx[b]], rows[b], gsem[b]).wait()
        pltpu.sync_copy(rows[b], msg_sh.at[ridx[b]], add=True)

    plsc.subcore_barrier()

    pltpu.sync_copy(msg_sh.at[pl.ds(sid * ROWS_PER_TILE, ROWS_PER_TILE)],
                    out.at[cid, pl.ds(sid * ROWS_PER_TILE, ROWS_PER_TILE)])


_sc_aggregate = functools.partial(
    pl.kernel,
    out_type=jax.ShapeDtypeStruct((NC, NPAD, D), jnp.float32),
    mesh=plsc.VectorSubcoreMesh(core_axis_name="c", subcore_axis_name="s",
                                num_cores=NC, num_subcores=NS),
    scratch_types=[
        [pltpu.VMEM((K,), jnp.int32) for _ in range(NBUF)],
        [pltpu.VMEM((K,), jnp.int32) for _ in range(NBUF)],
        [pltpu.VMEM((K, D), jnp.float32) for _ in range(NBUF)],
        pltpu.VMEM((16, D), jnp.float32),
        pltpu.VMEM_SHARED((NPAD, D), jnp.float32),
        [pltpu.SemaphoreType.DMA for _ in range(NBUF)],
        [pltpu.SemaphoreType.DMA for _ in range(NBUF)],
    ],
)(_sc_aggregate_body)


def _lstm_body(msgp_ref, h0_ref, c0_ref, wih_ref, whh_ref, bih_ref, bhh_ref,
               h_ref, c_ref):
    msg = msgp_ref[0] + msgp_ref[1]
    dn = (((1,), (1,)), ((), ()))
    gates = (lax.dot_general(msg, wih_ref[...], dn,
                             preferred_element_type=jnp.float32)
             + lax.dot_general(h0_ref[...], whh_ref[...], dn,
                               preferred_element_type=jnp.float32)
             + bih_ref[...] + bhh_ref[...])
    i = jax.nn.sigmoid(gates[:, 0 * D:1 * D])
    f = jax.nn.sigmoid(gates[:, 1 * D:2 * D])
    g = jnp.tanh(gates[:, 2 * D:3 * D])
    o = jax.nn.sigmoid(gates[:, 3 * D:4 * D])
    c_new = f * c0_ref[...] + i * g
    c_ref[...] = c_new
    h_ref[...] = o * jnp.tanh(c_new)


def _lstm_call(msgp, h0, c0, wih, whh, bih, bhh):
    R = 1000
    grid = N // R
    return pl.pallas_call(
        _lstm_body,
        grid=(grid,),
        in_specs=[
            pl.BlockSpec((NC, R, D), lambda i: (0, i, 0)),
            pl.BlockSpec((R, D), lambda i: (i, 0)),
            pl.BlockSpec((R, D), lambda i: (i, 0)),
            pl.BlockSpec((4 * D, D), lambda i: (0, 0)),
            pl.BlockSpec((4 * D, D), lambda i: (0, 0)),
            pl.BlockSpec((1, 4 * D), lambda i: (0, 0)),
            pl.BlockSpec((1, 4 * D), lambda i: (0, 0)),
        ],
        out_specs=[
            pl.BlockSpec((R, D), lambda i: (i, 0)),
            pl.BlockSpec((R, D), lambda i: (i, 0)),
        ],
        out_shape=[
            jax.ShapeDtypeStruct((N, D), jnp.float32),
            jax.ShapeDtypeStruct((N, D), jnp.float32),
        ],
    )(msgp, h0, c0, wih, whh, bih, bhh)


def kernel(edge_index, x_l, h0, c0, W_ih, W_hh, b_ih, b_hh):
    ei = edge_index.astype(jnp.int32)
    pad = EPAD - E
    dummy_rows = N + jnp.arange(pad, dtype=jnp.int32) % (NPAD - N)
    rowp = jnp.concatenate([ei[0], dummy_rows])
    colp = jnp.concatenate([ei[1], jnp.zeros((pad,), jnp.int32)])

    msgp = _sc_aggregate(rowp, colp, x_l)

    h_new, c_new = _lstm_call(msgp, h0, c0, W_ih, W_hh,
                              b_ih.reshape(1, 4 * D), b_hh.reshape(1, 4 * D))
    return (h_new, c_new)


# speedup vs baseline: 8.7045x; 1.1440x over previous
import functools

import jax
import jax.numpy as jnp
from jax import lax
from jax.experimental import pallas as pl
from jax.experimental.pallas import tpu as pltpu
from jax.experimental.pallas import tpu_sc as plsc

D = 128
N = 10000
E = 320000
NC, NS = 2, 16
K = 128
NBUF = 3
NCHUNK = E // K
CPW_A = 123
CPW_B = 33
NEXTRA = NCHUNK - NS * (CPW_A + CPW_B)
ROWS_PER_TILE = 632
NPAD = NS * ROWS_PER_TILE


def _sc_aggregate_body(ei, xl, out, cidx, ridx, rows, msg_sh, isem, gsem):
    cid = lax.axis_index("c")
    sid = lax.axis_index("s")
    cpw = jnp.where(cid == 0, CPW_A, CPW_B)
    c0 = jnp.where(cid == 0, sid * CPW_A, NS * CPW_A + sid * CPW_B)
    steady_trip = jnp.where(cid == 0, (CPW_A - 3) // 3, (CPW_B - 3) // 3)

    zero = jnp.zeros((16,), jnp.float32)

    def zrow(r, carry):
        for l in range(D // 16):
            rows[0][r, pl.ds(l * 16, 16)] = zero
        return carry
    lax.fori_loop(0, K, zrow, 0)
    for c in range(ROWS_PER_TILE // K):
        pltpu.sync_copy(rows[0],
                        msg_sh.at[pl.ds(sid * ROWS_PER_TILE + c * K, K)])
    rem = ROWS_PER_TILE % K
    if rem:
        base = sid * ROWS_PER_TILE + (ROWS_PER_TILE // K) * K
        pltpu.sync_copy(rows[0].at[pl.ds(0, rem)],
                        msg_sh.at[pl.ds(base, rem)])
    plsc.subcore_barrier()

    def load_idx(j, b):
        pltpu.async_copy(ei.at[1, pl.ds(j * K, K)], cidx[b], isem[b])
        pltpu.async_copy(ei.at[0, pl.ds(j * K, K)], ridx[b], isem[b])

    def wait_idx(j, b):
        pltpu.make_async_copy(ei.at[1, pl.ds(j * K, K)], cidx[b],
                              isem[b]).wait()
        pltpu.make_async_copy(ei.at[0, pl.ds(j * K, K)], ridx[b],
                              isem[b]).wait()

    def seg(j, b, b2):
        wait_idx(j + 2, b2)
        pltpu.async_copy(xl.at[cidx[b2]], rows[b2], gsem[b2])
        pltpu.make_async_copy(xl.at[cidx[b]], rows[b], gsem[b]).wait()
        pltpu.sync_copy(rows[b], msg_sh.at[ridx[b]], add=True)
        load_idx(j + 3, b)

    for b in range(2):
        load_idx(c0 + b, b)
    for b in range(2):
        wait_idx(c0 + b, b)
        pltpu.async_copy(xl.at[cidx[b]], rows[b], gsem[b])
    load_idx(c0 + 2, 2)

    def acc_body(t, carry):
        j = c0 + 3 * t
        for b in range(3):
            seg(j + b, b, (b + 2) % 3)
        return carry
    lax.fori_loop(0, steady_trip, acc_body, 0)

    wait_idx(c0 + cpw - 1, 2)
    pltpu.async_copy(xl.at[cidx[2]], rows[2], gsem[2])
    for b in range(3):
        pltpu.make_async_copy(xl.at[cidx[b]], rows[b], gsem[b]).wait()
        pltpu.sync_copy(rows[b], msg_sh.at[ridx[b]], add=True)

    @pl.when(jnp.logical_and(cid == 0, sid < NEXTRA))
    def _extra():
        j = NS * CPW_A + NS * CPW_B + sid
        load_idx(j, 0)
        wait_idx(j, 0)
        pltpu.async_copy(xl.at[cidx[0]], rows[0], gsem[0])
        pltpu.make_async_copy(xl.at[cidx[0]], rows[0], gsem[0]).wait()
        pltpu.sync_copy(rows[0], msg_sh.at[ridx[0]], add=True)

    plsc.subcore_barrier()

    pltpu.sync_copy(msg_sh.at[pl.ds(sid * ROWS_PER_TILE, ROWS_PER_TILE)],
                    out.at[cid, pl.ds(sid * ROWS_PER_TILE, ROWS_PER_TILE)])


_sc_aggregate = functools.partial(
    pl.kernel,
    out_type=jax.ShapeDtypeStruct((NC, NPAD, D), jnp.float32),
    mesh=plsc.VectorSubcoreMesh(core_axis_name="c", subcore_axis_name="s",
                                num_cores=NC, num_subcores=NS),
    scratch_types=[
        [pltpu.VMEM((K,), jnp.int32) for _ in range(NBUF)],
        [pltpu.VMEM((K,), jnp.int32) for _ in range(NBUF)],
        [pltpu.VMEM((K, D), jnp.float32) for _ in range(NBUF)],
        pltpu.VMEM_SHARED((NPAD, D), jnp.float32),
        [pltpu.SemaphoreType.DMA for _ in range(NBUF)],
        [pltpu.SemaphoreType.DMA for _ in range(NBUF)],
    ],
)(_sc_aggregate_body)


def _lstm_body(msgp_ref, h0_ref, c0_ref, wih_ref, whh_ref, bih_ref, bhh_ref,
               h_ref, c_ref):
    msg = msgp_ref[0] + msgp_ref[1]
    dn = (((1,), (1,)), ((), ()))
    gates = (lax.dot_general(msg, wih_ref[...], dn,
                             preferred_element_type=jnp.float32)
             + lax.dot_general(h0_ref[...], whh_ref[...], dn,
                               preferred_element_type=jnp.float32)
             + bih_ref[...] + bhh_ref[...])
    i = jax.nn.sigmoid(gates[:, 0 * D:1 * D])
    f = jax.nn.sigmoid(gates[:, 1 * D:2 * D])
    g = jnp.tanh(gates[:, 2 * D:3 * D])
    o = jax.nn.sigmoid(gates[:, 3 * D:4 * D])
    c_new = f * c0_ref[...] + i * g
    c_ref[...] = c_new
    h_ref[...] = o * jnp.tanh(c_new)


def _lstm_call(msgp, h0, c0, wih, whh, bih, bhh):
    R = 1000
    grid = N // R
    return pl.pallas_call(
        _lstm_body,
        grid=(grid,),
        in_specs=[
            pl.BlockSpec((NC, R, D), lambda i: (0, i, 0)),
            pl.BlockSpec((R, D), lambda i: (i, 0)),
            pl.BlockSpec((R, D), lambda i: (i, 0)),
            pl.BlockSpec((4 * D, D), lambda i: (0, 0)),
            pl.BlockSpec((4 * D, D), lambda i: (0, 0)),
            pl.BlockSpec((1, 4 * D), lambda i: (0, 0)),
            pl.BlockSpec((1, 4 * D), lambda i: (0, 0)),
        ],
        out_specs=[
            pl.BlockSpec((R, D), lambda i: (i, 0)),
            pl.BlockSpec((R, D), lambda i: (i, 0)),
        ],
        out_shape=[
            jax.ShapeDtypeStruct((N, D), jnp.float32),
            jax.ShapeDtypeStruct((N, D), jnp.float32),
        ],
    )(msgp, h0, c0, wih, whh, bih, bhh)


def kernel(edge_index, x_l, h0, c0, W_ih, W_hh, b_ih, b_hh):
    ei = edge_index.astype(jnp.int32)

    msgp = _sc_aggregate(ei, x_l)

    h_new, c_new = _lstm_call(msgp, h0, c0, W_ih, W_hh,
                              b_ih.reshape(1, 4 * D), b_hh.reshape(1, 4 * D))
    return (h_new, c_new)


# speedup vs baseline: 11.7811x; 1.3534x over previous
import functools

import jax
import jax.numpy as jnp
from jax import lax
from jax.experimental import pallas as pl
from jax.experimental.pallas import tpu as pltpu
from jax.experimental.pallas import tpu_sc as plsc

D = 128
N = 10000
E = 320000
NC, NS = 2, 16
K = 128
NBUF = 3
NCHUNK = E // K
CPW_A = 78
CPW_B = 78
NEXTRA = NCHUNK - NS * (CPW_A + CPW_B)
ROWS_PER_TILE = 632
NPAD = NS * ROWS_PER_TILE


def _sc_aggregate_body(ei, xl, out, cidx, ridx, rows, msg_sh, isem, gsem):
    cid = lax.axis_index("c")
    sid = lax.axis_index("s")
    cpw = jnp.where(cid == 0, CPW_A, CPW_B)
    c0 = jnp.where(cid == 0, sid * CPW_A, NS * CPW_A + sid * CPW_B)
    steady_trip = jnp.where(cid == 0, (CPW_A - 3) // 3, (CPW_B - 3) // 3)

    zero = jnp.zeros((16,), jnp.float32)

    def zrow(r, carry):
        for l in range(D // 16):
            rows[0][r, pl.ds(l * 16, 16)] = zero
        return carry
    lax.fori_loop(0, K, zrow, 0)
    for c in range(ROWS_PER_TILE // K):
        pltpu.sync_copy(rows[0],
                        msg_sh.at[pl.ds(sid * ROWS_PER_TILE + c * K, K)])
    rem = ROWS_PER_TILE % K
    if rem:
        base = sid * ROWS_PER_TILE + (ROWS_PER_TILE // K) * K
        pltpu.sync_copy(rows[0].at[pl.ds(0, rem)],
                        msg_sh.at[pl.ds(base, rem)])
    plsc.subcore_barrier()

    def load_idx(j, b):
        pltpu.async_copy(ei.at[1, pl.ds(j * K, K)], cidx[b], isem[b])
        pltpu.async_copy(ei.at[0, pl.ds(j * K, K)], ridx[b], isem[b])

    def wait_idx(j, b):
        pltpu.make_async_copy(ei.at[1, pl.ds(j * K, K)], cidx[b],
                              isem[b]).wait()
        pltpu.make_async_copy(ei.at[0, pl.ds(j * K, K)], ridx[b],
                              isem[b]).wait()

    def seg(j, b, b2):
        wait_idx(j + 2, b2)
        pltpu.async_copy(xl.at[cidx[b2]], rows[b2], gsem[b2])
        pltpu.make_async_copy(xl.at[cidx[b]], rows[b], gsem[b]).wait()
        pltpu.sync_copy(rows[b], msg_sh.at[ridx[b]], add=True)
        load_idx(j + 3, b)

    for b in range(2):
        load_idx(c0 + b, b)
    for b in range(2):
        wait_idx(c0 + b, b)
        pltpu.async_copy(xl.at[cidx[b]], rows[b], gsem[b])
    load_idx(c0 + 2, 2)

    def acc_body(t, carry):
        j = c0 + 3 * t
        for b in range(3):
            seg(j + b, b, (b + 2) % 3)
        return carry
    lax.fori_loop(0, steady_trip, acc_body, 0)

    wait_idx(c0 + cpw - 1, 2)
    pltpu.async_copy(xl.at[cidx[2]], rows[2], gsem[2])
    for b in range(3):
        pltpu.make_async_copy(xl.at[cidx[b]], rows[b], gsem[b]).wait()
        pltpu.sync_copy(rows[b], msg_sh.at[ridx[b]], add=True)

    @pl.when(jnp.logical_and(cid == 0, sid < NEXTRA))
    def _extra():
        j = NS * CPW_A + NS * CPW_B + sid
        load_idx(j, 0)
        wait_idx(j, 0)
        pltpu.async_copy(xl.at[cidx[0]], rows[0], gsem[0])
        pltpu.make_async_copy(xl.at[cidx[0]], rows[0], gsem[0]).wait()
        pltpu.sync_copy(rows[0], msg_sh.at[ridx[0]], add=True)

    plsc.subcore_barrier()

    pltpu.sync_copy(msg_sh.at[pl.ds(sid * ROWS_PER_TILE, ROWS_PER_TILE)],
                    out.at[cid, pl.ds(sid * ROWS_PER_TILE, ROWS_PER_TILE)])


_sc_aggregate = functools.partial(
    pl.kernel,
    out_type=jax.ShapeDtypeStruct((NC, NPAD, D), jnp.float32),
    mesh=plsc.VectorSubcoreMesh(core_axis_name="c", subcore_axis_name="s",
                                num_cores=NC, num_subcores=NS),
    scratch_types=[
        [pltpu.VMEM((K,), jnp.int32) for _ in range(NBUF)],
        [pltpu.VMEM((K,), jnp.int32) for _ in range(NBUF)],
        [pltpu.VMEM((K, D), jnp.float32) for _ in range(NBUF)],
        pltpu.VMEM_SHARED((NPAD, D), jnp.float32),
        [pltpu.SemaphoreType.DMA for _ in range(NBUF)],
        [pltpu.SemaphoreType.DMA for _ in range(NBUF)],
    ],
)(_sc_aggregate_body)


def _lstm_body(msgp_ref, h0_ref, c0_ref, wih_ref, whh_ref, bih_ref, bhh_ref,
               h_ref, c_ref):
    msg = msgp_ref[0] + msgp_ref[1]
    dn = (((1,), (1,)), ((), ()))
    gates = (lax.dot_general(msg, wih_ref[...], dn,
                             preferred_element_type=jnp.float32)
             + lax.dot_general(h0_ref[...], whh_ref[...], dn,
                               preferred_element_type=jnp.float32)
             + bih_ref[...] + bhh_ref[...])
    i = jax.nn.sigmoid(gates[:, 0 * D:1 * D])
    f = jax.nn.sigmoid(gates[:, 1 * D:2 * D])
    g = jnp.tanh(gates[:, 2 * D:3 * D])
    o = jax.nn.sigmoid(gates[:, 3 * D:4 * D])
    c_new = f * c0_ref[...] + i * g
    c_ref[...] = c_new
    h_ref[...] = o * jnp.tanh(c_new)


def _lstm_call(msgp, h0, c0, wih, whh, bih, bhh):
    R = 1000
    grid = N // R
    return pl.pallas_call(
        _lstm_body,
        grid=(grid,),
        in_specs=[
            pl.BlockSpec((NC, R, D), lambda i: (0, i, 0)),
            pl.BlockSpec((R, D), lambda i: (i, 0)),
            pl.BlockSpec((R, D), lambda i: (i, 0)),
            pl.BlockSpec((4 * D, D), lambda i: (0, 0)),
            pl.BlockSpec((4 * D, D), lambda i: (0, 0)),
            pl.BlockSpec((1, 4 * D), lambda i: (0, 0)),
            pl.BlockSpec((1, 4 * D), lambda i: (0, 0)),
        ],
        out_specs=[
            pl.BlockSpec((R, D), lambda i: (i, 0)),
            pl.BlockSpec((R, D), lambda i: (i, 0)),
        ],
        out_shape=[
            jax.ShapeDtypeStruct((N, D), jnp.float32),
            jax.ShapeDtypeStruct((N, D), jnp.float32),
        ],
    )(msgp, h0, c0, wih, whh, bih, bhh)


def kernel(edge_index, x_l, h0, c0, W_ih, W_hh, b_ih, b_hh):
    ei = edge_index.astype(jnp.int32)

    msgp = _sc_aggregate(ei, x_l)

    h_new, c_new = _lstm_call(msgp, h0, c0, W_ih, W_hh,
                              b_ih.reshape(1, 4 * D), b_hh.reshape(1, 4 * D))
    return (h_new, c_new)


# speedup vs baseline: 13.4620x; 1.1427x over previous
import functools

import jax
import jax.numpy as jnp
from jax import lax
from jax.experimental import pallas as pl
from jax.experimental.pallas import tpu as pltpu
from jax.experimental.pallas import tpu_sc as plsc

D = 128
N = 10000
E = 320000
NC, NS = 2, 16
K = 128
NCHUNK = E // K
CPT = 78
NEXTRA = NCHUNK - NC * NS * CPT
ROWS_PER_TILE = 632
NPAD = NS * ROWS_PER_TILE


def _sc_aggregate_body(ei, xl, out, cidx, ridx, rows, msg_sh,
                       isem, rsem, gsem, ssem):
    cid = lax.axis_index("c")
    sid = lax.axis_index("s")
    c0 = (cid * NS + sid) * CPT

    zero = jnp.zeros((16,), jnp.float32)

    def zrow(r, carry):
        for l in range(D // 16):
            rows[0][r, pl.ds(l * 16, 16)] = zero
        return carry
    lax.fori_loop(0, K, zrow, 0)
    for c in range(ROWS_PER_TILE // K):
        pltpu.sync_copy(rows[0],
                        msg_sh.at[pl.ds(sid * ROWS_PER_TILE + c * K, K)])
    rem = ROWS_PER_TILE % K
    if rem:
        base = sid * ROWS_PER_TILE + (ROWS_PER_TILE // K) * K
        pltpu.sync_copy(rows[0].at[pl.ds(0, rem)],
                        msg_sh.at[pl.ds(base, rem)])
    plsc.subcore_barrier()

    def load_cidx(j, b):
        pltpu.async_copy(ei.at[1, pl.ds(j * K, K)], cidx[b], isem[b])

    def wait_cidx(j, b):
        pltpu.make_async_copy(ei.at[1, pl.ds(j * K, K)], cidx[b],
                              isem[b]).wait()

    def load_ridx(j, b):
        pltpu.async_copy(ei.at[0, pl.ds(j * K, K)], ridx[b], rsem[b])

    def wait_ridx(j, b):
        pltpu.make_async_copy(ei.at[0, pl.ds(j * K, K)], ridx[b],
                              rsem[b]).wait()

    def wait_gather(b):
        pltpu.make_async_copy(xl.at[cidx[b]], rows[b], gsem[b]).wait()

    def wait_scatter(b):
        pltpu.make_async_copy(rows[b], msg_sh.at[ridx[b]], ssem[b]).wait()

    def seg_sync(j, o):
        b3, b2 = o % 3, (o + 2) % 3
        wait_cidx(j + 2, b2)
        load_ridx(j + 2, b2)
        pltpu.async_copy(xl.at[cidx[b2]], rows[b2], gsem[b2])
        wait_gather(b3)
        wait_ridx(j, b3)
        pltpu.sync_copy(rows[b3], msg_sh.at[ridx[b3]], add=True)
        load_cidx(j + 3, b3)

    def seg_async(j, o, first=False):
        b3, b2 = o % 3, (o + 2) % 3
        wait_cidx(j + 2, b2)
        if not first:
            wait_scatter(b2)
        load_ridx(j + 2, b2)
        pltpu.async_copy(xl.at[cidx[b2]], rows[b2], gsem[b2])
        wait_gather(b3)
        wait_ridx(j, b3)
        pltpu.async_copy(rows[b3], msg_sh.at[ridx[b3]], ssem[b3], add=True)
        load_cidx(j + 3, b3)

    for o in range(2):
        load_cidx(c0 + o, o)
        load_ridx(c0 + o, o)
    for o in range(2):
        wait_cidx(c0 + o, o)
        pltpu.async_copy(xl.at[cidx[o]], rows[o], gsem[o])
    load_cidx(c0 + 2, 2)
    for o in range(3):
        seg_sync(c0 + o, o)

    for s in range(6):
        seg_async(c0 + 3 + s, 3 + s, first=(s == 0))

    def acc_body(t, carry):
        j = c0 + 9 + 6 * t
        for s in range(6):
            seg_async(j + s, 9 + s)
        return carry
    lax.fori_loop(0, (CPT - 9 - 3) // 6, acc_body, 0)

    bl = (CPT - 1) % 3
    wait_cidx(c0 + CPT - 1, bl)
    wait_scatter(bl)
    load_ridx(c0 + CPT - 1, bl)
    pltpu.async_copy(xl.at[cidx[bl]], rows[bl], gsem[bl])
    for s in range(3):
        o = CPT - 3 + s
        b = o % 3
        wait_gather(b)
        wait_ridx(c0 + o, b)
        pltpu.sync_copy(rows[b], msg_sh.at[ridx[b]], add=True)

    @pl.when(jnp.logical_and(cid == 0, sid < NEXTRA))
    def _extra():
        j = NC * NS * CPT + sid
        load_cidx(j, 0)
        load_ridx(j, 0)
        wait_cidx(j, 0)
        wait_ridx(j, 0)
        pltpu.async_copy(xl.at[cidx[0]], rows[0], gsem[0])
        wait_gather(0)
        pltpu.sync_copy(rows[0], msg_sh.at[ridx[0]], add=True)

    plsc.subcore_barrier()

    pltpu.sync_copy(msg_sh.at[pl.ds(sid * ROWS_PER_TILE, ROWS_PER_TILE)],
                    out.at[cid, pl.ds(sid * ROWS_PER_TILE, ROWS_PER_TILE)])


_sc_aggregate = functools.partial(
    pl.kernel,
    out_type=jax.ShapeDtypeStruct((NC, NPAD, D), jnp.float32),
    mesh=plsc.VectorSubcoreMesh(core_axis_name="c", subcore_axis_name="s",
                                num_cores=NC, num_subcores=NS),
    scratch_types=[
        [pltpu.VMEM((K,), jnp.int32) for _ in range(3)],
        [pltpu.VMEM((K,), jnp.int32) for _ in range(3)],
        [pltpu.VMEM((K, D), jnp.float32) for _ in range(3)],
        pltpu.VMEM_SHARED((NPAD, D), jnp.float32),
        [pltpu.SemaphoreType.DMA for _ in range(3)],
        [pltpu.SemaphoreType.DMA for _ in range(3)],
        [pltpu.SemaphoreType.DMA for _ in range(3)],
        [pltpu.SemaphoreType.DMA for _ in range(3)],
    ],
)(_sc_aggregate_body)


def _lstm_body(msgp_ref, h0_ref, c0_ref, wih_ref, whh_ref, bih_ref, bhh_ref,
               h_ref, c_ref):
    msg = msgp_ref[0] + msgp_ref[1]
    dn = (((1,), (1,)), ((), ()))
    gates = (lax.dot_general(msg, wih_ref[...], dn,
                             preferred_element_type=jnp.float32)
             + lax.dot_general(h0_ref[...], whh_ref[...], dn,
                               preferred_element_type=jnp.float32)
             + bih_ref[...] + bhh_ref[...])
    i = jax.nn.sigmoid(gates[:, 0 * D:1 * D])
    f = jax.nn.sigmoid(gates[:, 1 * D:2 * D])
    g = jnp.tanh(gates[:, 2 * D:3 * D])
    o = jax.nn.sigmoid(gates[:, 3 * D:4 * D])
    c_new = f * c0_ref[...] + i * g
    c_ref[...] = c_new
    h_ref[...] = o * jnp.tanh(c_new)


def _lstm_call(msgp, h0, c0, wih, whh, bih, bhh):
    R = 1000
    grid = N // R
    return pl.pallas_call(
        _lstm_body,
        grid=(grid,),
        in_specs=[
            pl.BlockSpec((NC, R, D), lambda i: (0, i, 0)),
            pl.BlockSpec((R, D), lambda i: (i, 0)),
            pl.BlockSpec((R, D), lambda i: (i, 0)),
            pl.BlockSpec((4 * D, D), lambda i: (0, 0)),
            pl.BlockSpec((4 * D, D), lambda i: (0, 0)),
            pl.BlockSpec((1, 4 * D), lambda i: (0, 0)),
            pl.BlockSpec((1, 4 * D), lambda i: (0, 0)),
        ],
        out_specs=[
            pl.BlockSpec((R, D), lambda i: (i, 0)),
            pl.BlockSpec((R, D), lambda i: (i, 0)),
        ],
        out_shape=[
            jax.ShapeDtypeStruct((N, D), jnp.float32),
            jax.ShapeDtypeStruct((N, D), jnp.float32),
        ],
    )(msgp, h0, c0, wih, whh, bih, bhh)


def kernel(edge_index, x_l, h0, c0, W_ih, W_hh, b_ih, b_hh):
    ei = edge_index.astype(jnp.int32)

    msgp = _sc_aggregate(ei, x_l)

    h_new, c_new = _lstm_call(msgp, h0, c0, W_ih, W_hh,
                              b_ih.reshape(1, 4 * D), b_hh.reshape(1, 4 * D))
    return (h_new, c_new)
